# baseline (speedup 1.0000x reference)
import functools

import jax
import jax.numpy as jnp
from jax import lax
from jax.experimental import pallas as pl
from jax.experimental.pallas import tpu as pltpu
from jax.experimental.pallas import tpu_sc as plsc

DIM = 128
NW = 32
R = 128
POS_ROWS = 513


def _build(n_rows):
    per_w = n_rows // NW
    steps = per_w // R
    mesh = plsc.VectorSubcoreMesh(core_axis_name="c", subcore_axis_name="s")

    @functools.partial(
        pl.kernel,
        mesh=mesh,
        out_type=jax.ShapeDtypeStruct((n_rows, DIM), jnp.float32),
        scratch_types=[
            pltpu.VMEM((steps, R), jnp.int32),
            pltpu.VMEM((R,), jnp.int32),
            pltpu.VMEM((POS_ROWS * DIM,), jnp.float32),
            pltpu.VMEM((R, DIM), jnp.float32),
            pltpu.SemaphoreType.DMA,
        ],
    )
    def k(tok_idx_hbm, pos_off_hbm, emb_hbm, pos_flat_hbm, out_hbm,
          idx_tok, pidx, pos_flat, tok_buf, sem):
        wid = lax.axis_index("s") * 2 + lax.axis_index("c")
        wbase = wid * per_w

        pltpu.sync_copy(pos_flat_hbm, pos_flat)
        pltpu.sync_copy(tok_idx_hbm.at[wid], idx_tok)

        def step(si, carry):
            pltpu.sync_copy(pos_off_hbm.at[wid * steps + si], pidx)
            pltpu.async_copy(emb_hbm.at[idx_tok.at[si]], tok_buf, sem).wait()

            def add_group(g, c):
                offv = pidx[pl.ds(g * 16, 16)]
                for t in range(16):
                    off = offv[t]
                    for j in range(DIM // 16):
                        sl = pl.ds(j * 16, 16)
                        row = g * 16 + t
                        tok_buf[row, sl] = (
                            tok_buf[row, sl] + pos_flat[pl.ds(off + j * 16, 16)])
                return c

            lax.fori_loop(0, R // 16, add_group, 0)
            pltpu.sync_copy(tok_buf, out_hbm.at[pl.ds(wbase + si * R, R)])
            return carry

        lax.fori_loop(0, steps, step, 0)

    return k


@jax.jit
def kernel(input_tensor, incremental_mask, emb_table, pos_table):
    b, l = input_tensor.shape
    n = b * l
    per_w = n // NW
    steps = per_w // R
    tok_idx = input_tensor.reshape(NW, steps, R)
    pos_off = (incremental_mask * DIM).reshape(NW * steps, R)
    pos_flat = pos_table.reshape(POS_ROWS * DIM)
    out = _build(n)(tok_idx, pos_off, emb_table, pos_flat)
    return out.reshape(b, l, DIM)


# speedup vs baseline: 2.0076x; 2.0076x over previous
import functools

import jax
import jax.numpy as jnp
from jax import lax
from jax.experimental import pallas as pl
from jax.experimental.pallas import tpu as pltpu
from jax.experimental.pallas import tpu_sc as plsc

DIM = 128
NW = 32
R = 128


def _build(n_rows):
    per_w = n_rows // NW
    steps = per_w // R
    assert steps % 2 == 0
    mesh = plsc.VectorSubcoreMesh(core_axis_name="c", subcore_axis_name="s")

    @functools.partial(
        pl.kernel,
        mesh=mesh,
        out_type=jax.ShapeDtypeStruct((n_rows, DIM), jnp.float32),
        scratch_types=[
            pltpu.VMEM((steps, R), jnp.int32),
            pltpu.VMEM((steps, R), jnp.int32),
            pltpu.VMEM((R, DIM), jnp.float32),
            pltpu.VMEM((R, DIM), jnp.float32),
            pltpu.VMEM((R, DIM), jnp.float32),
            pltpu.VMEM((R, DIM), jnp.float32),
            pltpu.SemaphoreType.DMA,
            pltpu.SemaphoreType.DMA,
            pltpu.SemaphoreType.DMA,
            pltpu.SemaphoreType.DMA,
        ],
    )
    def k(tok_idx_hbm, pos_idx_hbm, emb_hbm, pos_hbm, out_hbm,
          idx_tok, idx_pos, tok0, tok1, pos0, pos1,
          sem_g0, sem_g1, sem_o0, sem_o1):
        wid = lax.axis_index("s") * 2 + lax.axis_index("c")
        wbase = wid * per_w
        tok_b = (tok0, tok1)
        pos_b = (pos0, pos1)
        sem_g = (sem_g0, sem_g1)
        sem_o = (sem_o0, sem_o1)

        pltpu.sync_copy(tok_idx_hbm.at[wid], idx_tok)
        pltpu.sync_copy(pos_idx_hbm.at[wid], idx_pos)

        def issue(si, p):
            pltpu.async_copy(emb_hbm.at[idx_tok.at[si]], tok_b[p], sem_g[p])
            pltpu.async_copy(pos_hbm.at[idx_pos.at[si]], pos_b[p], sem_g[p])

        def wait_gathers(si, p):
            pltpu.make_async_copy(emb_hbm.at[idx_tok.at[si]], tok_b[p], sem_g[p]).wait()
            pltpu.make_async_copy(pos_hbm.at[idx_pos.at[si]], pos_b[p], sem_g[p]).wait()

        def add(p):
            tb, pb = tok_b[p], pos_b[p]

            def add_row(r, c):
                for j in range(DIM // 16):
                    sl = pl.ds(j * 16, 16)
                    tb[r, sl] = tb[r, sl] + pb[r, sl]
                return c

            lax.fori_loop(0, R, add_row, 0)

        def start_out(si, p):
            pltpu.async_copy(tok_b[p], out_hbm.at[pl.ds(wbase + si * R, R)], sem_o[p])

        def wait_out(si, p):
            pltpu.make_async_copy(
                tok_b[p], out_hbm.at[pl.ds(wbase + si * R, R)], sem_o[p]).wait()

        issue(0, 0)

        def body(i2, carry):
            i0 = i2 * 2
            i1 = i0 + 1

            @pl.when(i2 > 0)
            def _():
                wait_out(i0 - 1, 1)

            issue(i1, 1)
            wait_gathers(i0, 0)
            add(0)
            start_out(i0, 0)

            @pl.when(i2 < steps // 2 - 1)
            def _():
                wait_out(i0, 0)
                issue(i0 + 2, 0)

            wait_gathers(i1, 1)
            add(1)
            start_out(i1, 1)
            return carry

        lax.fori_loop(0, steps // 2, body, 0)
        wait_out(steps - 2, 0)
        wait_out(steps - 1, 1)

    return k


@jax.jit
def kernel(input_tensor, incremental_mask, emb_table, pos_table):
    b, l = input_tensor.shape
    n = b * l
    per_w = n // NW
    steps = per_w // R
    tok_idx = input_tensor.reshape(NW, steps, R)
    pos_idx = incremental_mask.reshape(NW, steps, R)
    out = _build(n)(tok_idx, pos_idx, emb_table, pos_table)
    return out.reshape(b, l, DIM)


# speedup vs baseline: 2.0179x; 1.0051x over previous
import functools

import jax
import jax.numpy as jnp
from jax import lax
from jax.experimental import pallas as pl
from jax.experimental.pallas import tpu as pltpu
from jax.experimental.pallas import tpu_sc as plsc

DIM = 128
NW = 32
R = 128
UNROLL = 6


def _build(n_rows):
    per_w = n_rows // NW
    steps = per_w // R
    main = (steps // UNROLL) * UNROLL
    mesh = plsc.VectorSubcoreMesh(core_axis_name="c", subcore_axis_name="s")

    @functools.partial(
        pl.kernel,
        mesh=mesh,
        out_type=jax.ShapeDtypeStruct((n_rows, DIM), jnp.float32),
        scratch_types=[
            pltpu.VMEM((steps, R), jnp.int32),
            pltpu.VMEM((steps, R), jnp.int32),
            pltpu.VMEM((R, DIM), jnp.float32),
            pltpu.VMEM((R, DIM), jnp.float32),
            pltpu.VMEM((R, DIM), jnp.float32),
            pltpu.VMEM((R, DIM), jnp.float32),
            pltpu.VMEM((R, DIM), jnp.float32),
            pltpu.SemaphoreType.DMA,
            pltpu.SemaphoreType.DMA,
            pltpu.SemaphoreType.DMA,
            pltpu.SemaphoreType.DMA,
            pltpu.SemaphoreType.DMA,
        ],
    )
    def k(tok_idx_hbm, pos_idx_hbm, emb_hbm, pos_hbm, out_hbm,
          idx_tok, idx_pos, tok0, tok1, tok2, pos0, pos1,
          sem_g0, sem_g1, sem_o0, sem_o1, sem_o2):
        wid = lax.axis_index("s") * 2 + lax.axis_index("c")
        wbase = wid * per_w
        tok_b = (tok0, tok1, tok2)
        pos_b = (pos0, pos1)
        sem_g = (sem_g0, sem_g1)
        sem_o = (sem_o0, sem_o1, sem_o2)

        pltpu.sync_copy(tok_idx_hbm.at[wid], idx_tok)
        pltpu.sync_copy(pos_idx_hbm.at[wid], idx_pos)

        def issue(si, kn):
            pltpu.async_copy(emb_hbm.at[idx_tok.at[si]], tok_b[kn % 3], sem_g[kn % 2])
            pltpu.async_copy(pos_hbm.at[idx_pos.at[si]], pos_b[kn % 2], sem_g[kn % 2])

        def wait_gathers(si, kn):
            pltpu.make_async_copy(
                emb_hbm.at[idx_tok.at[si]], tok_b[kn % 3], sem_g[kn % 2]).wait()
            pltpu.make_async_copy(
                pos_hbm.at[idx_pos.at[si]], pos_b[kn % 2], sem_g[kn % 2]).wait()

        def add(kn):
            tb, pb = tok_b[kn % 3], pos_b[kn % 2]

            def add_row(r, c):
                for j in range(DIM // 16):
                    sl = pl.ds(j * 16, 16)
                    tb[r, sl] = tb[r, sl] + pb[r, sl]
                return c

            lax.fori_loop(0, R, add_row, 0)

        def start_out(si, kn):
            pltpu.async_copy(
                tok_b[kn % 3], out_hbm.at[pl.ds(wbase + si * R, R)], sem_o[kn % 3])

        def wait_out(si, kn):
            pltpu.make_async_copy(
                tok_b[kn % 3], out_hbm.at[pl.ds(wbase + si * R, R)],
                sem_o[kn % 3]).wait()

        def half(si, kn, first_pair=False):
            if not first_pair:
                wait_out(si - 2, kn + 1)
            issue(si + 1, kn + 1)
            wait_gathers(si, kn)
            add(kn)
            start_out(si, kn)

        issue(0, 0)
        half(0, 0, first_pair=True)
        half(1, 1, first_pair=True)

        def body(g, carry):
            base = 2 + g * UNROLL
            for kk in range(UNROLL):
                half(base + kk, (2 + kk) % UNROLL)
            return carry

        n_loop = (main - 2 - 4) // UNROLL
        lax.fori_loop(0, n_loop, body, 0)
        for s in range(2 + n_loop * UNROLL, steps - 1):
            half(s, s % UNROLL)

        s_last = steps - 1
        wait_out(s_last - 2, s_last + 1)
        wait_gathers(s_last, s_last % UNROLL)
        add(s_last % UNROLL)
        start_out(s_last, s_last % UNROLL)

        wait_out(steps - 2, steps - 2)
        wait_out(steps - 1, steps - 1)

    return k


@jax.jit
def kernel(input_tensor, incremental_mask, emb_table, pos_table):
    b, l = input_tensor.shape
    n = b * l
    per_w = n // NW
    steps = per_w // R
    tok_idx = input_tensor.reshape(NW, steps, R)
    pos_idx = incremental_mask.reshape(NW, steps, R)
    out = _build(n)(tok_idx, pos_idx, emb_table, pos_table)
    return out.reshape(b, l, DIM)
